# baseline (speedup 1.0000x reference)
import jax
import jax.numpy as jnp
from jax.experimental import pallas as pl
from jax.experimental.pallas import tpu as pltpu

_H = 15
_HW = _H * _H
_HWP = 256
_EPS = 1e-6
_P = 32


def _fused_kernel(xa_ref, fw_ref, w1_ref, b_ref, o1_ref, o2_ref, g_ref,
                  f_ref):
    fw = fw_ref[...]
    bias_col = b_ref[...]
    ones_b = jnp.ones((1, _HWP), jnp.bfloat16)
    ones_col = jnp.ones((_HWP, 1), jnp.float32)

    def feats_block(p):
        x = xa_ref[p]
        f12 = jax.lax.dot_general(fw, x, (((1,), (0,)), ((), ())),
                                  preferred_element_type=jnp.float32)
        f_ref[p] = jnp.maximum(f12, 0.0).astype(jnp.bfloat16)

    invs = [None] * (2 * _P)

    def corr_block(p):
        f1 = f_ref[p, :, :_HWP]
        f2 = f_ref[p, :, _HWP:]
        for d, (fa, fb) in enumerate(((f1, f2), (f2, f1))):
            r = jax.lax.dot_general(fa, fb, (((0,), (0,)), ((), ())),
                                    preferred_element_type=jnp.float32)
            rb = jnp.maximum(r, 0.0).astype(jnp.bfloat16)
            g_ref[2 * p + d] = rb
            q = rb * rb
            s = jax.lax.dot_general(ones_b, q, (((1,), (0,)), ((), ())),
                                    preferred_element_type=jnp.float32)
            invs[2 * p + d] = jax.lax.rsqrt(s + _EPS)

    parts = {}

    def products_block(p):
        g1 = g_ref[2 * p]
        g2 = g_ref[2 * p + 1]
        for f in range(16):
            wf = w1_ref[f]
            parts[(p, 0, f)] = jax.lax.dot_general(
                ones_b, g1 * wf, (((1,), (0,)), ((), ())),
                preferred_element_type=jnp.float32)
            parts[(p, 1, f)] = jax.lax.dot_general(
                ones_b, g2 * wf, (((1,), (0,)), ((), ())),
                preferred_element_type=jnp.float32)

    def finish_block(p):
        smat = jnp.concatenate(
            [parts[(p, 0, f)] for f in range(16)]
            + [parts[(p, 1, f)] for f in range(16)], axis=0)
        scale = jnp.concatenate(
            [jnp.broadcast_to(invs[2 * p], (16, _HWP)),
             jnp.broadcast_to(invs[2 * p + 1], (16, _HWP))], axis=0)
        tmat = smat * scale
        tot = jax.lax.dot_general(tmat, ones_col, (((1,), (0,)), ((), ())),
                                  preferred_element_type=jnp.float32)
        o1_ref[p] = (tot[:16] + bias_col).astype(o1_ref.dtype)
        o2_ref[p] = (tot[16:] + bias_col).astype(o2_ref.dtype)

    for p in range(_P):
        feats_block(p)
    corr_block(0)
    for p in range(_P):
        if p + 1 < _P:
            corr_block(p + 1)
        products_block(p)
        if p >= 1:
            finish_block(p - 1)
    finish_block(_P - 1)


def kernel(img1, img2, feat_w, feat_b, reg_w, reg_b):
    B = img1.shape[0]
    C = feat_w.shape[0]
    pad = _HWP - _HW
    x1 = jnp.pad(img1.reshape(B, 3, _HW), ((0, 0), (0, 0), (0, pad)))
    x2 = jnp.pad(img2.reshape(B, 3, _HW), ((0, 0), (0, 0), (0, pad)))
    ch = jnp.concatenate([x1, x2], axis=2)
    lane = jnp.arange(2 * _HWP) % _HWP
    ones_ch = jnp.broadcast_to((lane < _HW).astype(jnp.float32),
                               (B, 1, 2 * _HWP))
    xa = jnp.concatenate([ch, ones_ch], axis=1)

    fw = jnp.concatenate([feat_w, feat_b.reshape(C, 1)], axis=1)

    r4 = reg_w.reshape(_H, _H, _HW, 16)
    w1 = jnp.transpose(r4, (3, 1, 0, 2)).reshape(16, _HW, _HW)
    w1 = jnp.pad(w1, ((0, 0), (0, pad), (0, pad))).astype(jnp.bfloat16)

    b2 = reg_b.reshape(16, 1).astype(jnp.float32)

    out1, out2 = pl.pallas_call(
        _fused_kernel,
        out_shape=(jax.ShapeDtypeStruct((B, 16, 1), img1.dtype),
                   jax.ShapeDtypeStruct((B, 16, 1), img1.dtype)),
        grid=(B // _P,),
        in_specs=[pl.BlockSpec((_P, 4, 2 * _HWP), lambda i: (i, 0, 0)),
                  pl.BlockSpec((C, 4), lambda i: (0, 0)),
                  pl.BlockSpec((16, _HWP, _HWP), lambda i: (0, 0, 0)),
                  pl.BlockSpec((16, 1), lambda i: (0, 0))],
        out_specs=(pl.BlockSpec((_P, 16, 1), lambda i: (i, 0, 0)),
                   pl.BlockSpec((_P, 16, 1), lambda i: (i, 0, 0))),
        scratch_shapes=[pltpu.VMEM((2 * _P, _HWP, _HWP), jnp.bfloat16),
                        pltpu.VMEM((_P, _HWP, 2 * _HWP), jnp.bfloat16)],
        compiler_params=pltpu.CompilerParams(
            dimension_semantics=("parallel",)),
    )(xa, fw, w1, b2)
    return out1.reshape(B, 16), out2.reshape(B, 16)
